# baseline (speedup 1.0000x reference)
import jax
import jax.numpy as jnp
from jax import lax
from jax.experimental import pallas as pl
from jax.experimental.pallas import tpu as pltpu
from jax.experimental.pallas import tpu_sc as plsc

B = 8
C_IN = 96
C_OUT = 192
H = 224
W = 224
PLANE = H * W
N_PLANES = B * C_IN
NW = 32
PLANES_PER_W = N_PLANES // NW

_mesh = plsc.VectorSubcoreMesh(core_axis_name="c", subcore_axis_name="s")


def _body(x_hbm, out_hbm, buf, zbuf):
    cid = lax.axis_index("c")
    sid = lax.axis_index("s")
    wid = sid * 2 + cid

    zeros16 = jnp.zeros((16,), jnp.float32)

    def zinit(i, carry):
        zbuf[pl.ds(i * 16, 16)] = zeros16
        return carry

    lax.fori_loop(0, PLANE // 16, zinit, 0)

    base = wid * PLANES_PER_W
    for k in range(PLANES_PER_W):
        p = base + k
        pltpu.sync_copy(x_hbm.at[p], buf)
        pltpu.sync_copy(buf, out_hbm.at[2 * p])
        pltpu.sync_copy(zbuf, out_hbm.at[2 * p + 1])


def kernel(x):
    xr = x.reshape(N_PLANES, PLANE)
    fn = pl.kernel(
        _body,
        out_type=jax.ShapeDtypeStruct((2 * N_PLANES, PLANE), jnp.float32),
        mesh=_mesh,
        scratch_types=[
            pltpu.VMEM((PLANE,), jnp.float32),
            pltpu.VMEM((PLANE,), jnp.float32),
        ],
    )
    out = fn(xr)
    return out.reshape(B, C_OUT, H, W)
